# baseline (device time: 7514 ns/iter reference)
import jax
import jax.numpy as jnp
from jax import lax
from jax.experimental import pallas as pl
from jax.experimental.pallas import tpu as pltpu

N_GLOBAL = 1024
EPS = 1e-5
NCHUNK = 1


def kernel(x, gamma, beta):
    m, n = x.shape
    cm = m // NCHUNK
    r = cm // 128

    def body(
        x_hbm,
        g_hbm,
        b_hbm,
        out_ref,
        xbuf,
        gbuf,
        bbuf,
        stats_ref,
        recv_ref,
        in_sems,
        gb_sems,
        send_sems,
        recv_sems,
    ):
        my_x = lax.axis_index("x")
        my_y = lax.axis_index("y")
        nbr = (my_x, 1 - my_y)

        g_cp = pltpu.make_async_copy(g_hbm, gbuf, gb_sems.at[0])
        g_cp.start()
        b_cp = pltpu.make_async_copy(b_hbm, bbuf, gb_sems.at[1])
        b_cp.start()
        in_copies = []
        for c in range(NCHUNK):
            cp = pltpu.make_async_copy(
                x_hbm.at[c * cm : (c + 1) * cm, :], xbuf.at[c], in_sems.at[c]
            )
            cp.start()
            in_copies.append(cp)

        barrier_sem = pltpu.get_barrier_semaphore()
        pl.semaphore_signal(
            barrier_sem, inc=1, device_id=nbr, device_id_type=pl.DeviceIdType.MESH
        )

        rdmas = []
        for c in range(NCHUNK):
            in_copies[c].wait()
            xa = xbuf[c]
            s = jnp.sum(xa, axis=1)
            q = jnp.sum(xa * xa, axis=1)
            base = c * 2 * r
            stats_ref[base : base + r, :] = s.reshape(r, 128)
            stats_ref[base + r : base + 2 * r, :] = q.reshape(r, 128)
            if c == 0:
                pl.semaphore_wait(barrier_sem, 1)
            rdma = pltpu.make_async_remote_copy(
                src_ref=stats_ref.at[base : base + 2 * r],
                dst_ref=recv_ref.at[base : base + 2 * r],
                send_sem=send_sems.at[c],
                recv_sem=recv_sems.at[c],
                device_id=nbr,
                device_id_type=pl.DeviceIdType.MESH,
            )
            rdma.start()
            rdmas.append(rdma)

        lane = lax.broadcasted_iota(jnp.int32, (128, 128), 1)
        sub = lax.broadcasted_iota(jnp.int32, (128, 128), 0)
        eye = (lane == sub).astype(jnp.float32)
        g_cp.wait()
        b_cp.wait()
        gv = gbuf[0:1, :]
        bv = bbuf[0:1, :]

        for c in range(NCHUNK):
            rdmas[c].wait()
            base = c * 2 * r
            tot = stats_ref[base : base + 2 * r, :] + recv_ref[base : base + 2 * r, :]
            meanl = tot[0:r, :] / N_GLOBAL
            invl = lax.rsqrt(tot[r : 2 * r, :] / N_GLOBAL - meanl * meanl + EPS)
            ab = jnp.concatenate([invl, -meanl * invl], axis=0)
            tt = lax.dot_general(
                eye,
                ab,
                (((1,), (1,)), ((), ())),
                preferred_element_type=jnp.float32,
            )
            for a in range(r):
                inv_col = tt[:, a : a + 1]
                off_col = tt[:, r + a : r + a + 1]
                lo, hi = a * 128, (a + 1) * 128
                xa = xbuf[c, lo:hi, :]
                out_ref[c * cm + lo : c * cm + hi, :] = (
                    gv * (xa * inv_col + off_col) + bv
                )

    g2 = pltpu.with_memory_space_constraint(
        gamma.reshape(1, n), pltpu.MemorySpace.HBM
    )
    b2 = pltpu.with_memory_space_constraint(
        beta.reshape(1, n), pltpu.MemorySpace.HBM
    )
    x2 = pltpu.with_memory_space_constraint(x, pltpu.MemorySpace.HBM)

    return pl.pallas_call(
        body,
        out_shape=jax.ShapeDtypeStruct((m, n), x.dtype),
        in_specs=[
            pl.BlockSpec(memory_space=pl.ANY),
            pl.BlockSpec(memory_space=pl.ANY),
            pl.BlockSpec(memory_space=pl.ANY),
        ],
        out_specs=pl.BlockSpec(memory_space=pltpu.VMEM),
        scratch_shapes=[
            pltpu.VMEM((NCHUNK, cm, n), jnp.float32),
            pltpu.VMEM((1, n), jnp.float32),
            pltpu.VMEM((1, n), jnp.float32),
            pltpu.VMEM((16, 128), jnp.float32),
            pltpu.VMEM((16, 128), jnp.float32),
            pltpu.SemaphoreType.DMA((NCHUNK,)),
            pltpu.SemaphoreType.DMA((2,)),
            pltpu.SemaphoreType.DMA((NCHUNK,)),
            pltpu.SemaphoreType.DMA((NCHUNK,)),
        ],
        compiler_params=pltpu.CompilerParams(collective_id=0),
    )(x2, g2, b2)


# device time: 7353 ns/iter; 1.0219x vs baseline; 1.0219x over previous
import jax
import jax.numpy as jnp
from jax import lax
from jax.experimental import pallas as pl
from jax.experimental.pallas import tpu as pltpu

N_GLOBAL = 1024
EPS = 1e-5
NCHUNK = 2


def kernel(x, gamma, beta):
    m, n = x.shape
    cm = m // NCHUNK
    r = cm // 128

    def body(
        x_hbm,
        g_hbm,
        b_hbm,
        out_ref,
        xbuf,
        gbuf,
        bbuf,
        stats_ref,
        recv_ref,
        in_sems,
        gb_sems,
        send_sems,
        recv_sems,
    ):
        my_x = lax.axis_index("x")
        my_y = lax.axis_index("y")
        nbr = (my_x, 1 - my_y)

        g_cp = pltpu.make_async_copy(g_hbm, gbuf, gb_sems.at[0])
        g_cp.start()
        b_cp = pltpu.make_async_copy(b_hbm, bbuf, gb_sems.at[1])
        b_cp.start()
        in_copies = []
        for c in range(NCHUNK):
            cp = pltpu.make_async_copy(
                x_hbm.at[c * cm : (c + 1) * cm, :], xbuf.at[c], in_sems.at[c]
            )
            cp.start()
            in_copies.append(cp)

        barrier_sem = pltpu.get_barrier_semaphore()
        pl.semaphore_signal(
            barrier_sem, inc=1, device_id=nbr, device_id_type=pl.DeviceIdType.MESH
        )

        rdmas = []
        for c in range(NCHUNK):
            in_copies[c].wait()
            xa = xbuf[c]
            s = jnp.sum(xa, axis=1)
            q = jnp.sum(xa * xa, axis=1)
            base = c * 2 * r
            stats_ref[base : base + r, :] = s.reshape(r, 128)
            stats_ref[base + r : base + 2 * r, :] = q.reshape(r, 128)
            if c == 0:
                pl.semaphore_wait(barrier_sem, 1)
            rdma = pltpu.make_async_remote_copy(
                src_ref=stats_ref.at[base : base + 2 * r],
                dst_ref=recv_ref.at[base : base + 2 * r],
                send_sem=send_sems.at[c],
                recv_sem=recv_sems.at[c],
                device_id=nbr,
                device_id_type=pl.DeviceIdType.MESH,
            )
            rdma.start()
            rdmas.append(rdma)

        lane = lax.broadcasted_iota(jnp.int32, (128, 128), 1)
        sub = lax.broadcasted_iota(jnp.int32, (128, 128), 0)
        eye = (lane == sub).astype(jnp.float32)
        g_cp.wait()
        b_cp.wait()
        gv = gbuf[0:1, :]
        bv = bbuf[0:1, :]

        for c in range(NCHUNK):
            rdmas[c].wait()
            base = c * 2 * r
            tot = stats_ref[base : base + 2 * r, :] + recv_ref[base : base + 2 * r, :]
            meanl = tot[0:r, :] / N_GLOBAL
            invl = lax.rsqrt(tot[r : 2 * r, :] / N_GLOBAL - meanl * meanl + EPS)
            ab = jnp.concatenate([invl, -meanl * invl], axis=0)
            tt = lax.dot_general(
                eye,
                ab,
                (((1,), (1,)), ((), ())),
                preferred_element_type=jnp.float32,
            )
            for a in range(r):
                inv_col = tt[:, a : a + 1]
                off_col = tt[:, r + a : r + a + 1]
                lo, hi = a * 128, (a + 1) * 128
                xa = xbuf[c, lo:hi, :]
                out_ref[c * cm + lo : c * cm + hi, :] = (
                    gv * (xa * inv_col + off_col) + bv
                )

    g2 = pltpu.with_memory_space_constraint(
        gamma.reshape(1, n), pltpu.MemorySpace.HBM
    )
    b2 = pltpu.with_memory_space_constraint(
        beta.reshape(1, n), pltpu.MemorySpace.HBM
    )
    x2 = pltpu.with_memory_space_constraint(x, pltpu.MemorySpace.HBM)

    return pl.pallas_call(
        body,
        out_shape=jax.ShapeDtypeStruct((m, n), x.dtype),
        in_specs=[
            pl.BlockSpec(memory_space=pl.ANY),
            pl.BlockSpec(memory_space=pl.ANY),
            pl.BlockSpec(memory_space=pl.ANY),
        ],
        out_specs=pl.BlockSpec(memory_space=pltpu.VMEM),
        scratch_shapes=[
            pltpu.VMEM((NCHUNK, cm, n), jnp.float32),
            pltpu.VMEM((1, n), jnp.float32),
            pltpu.VMEM((1, n), jnp.float32),
            pltpu.VMEM((16, 128), jnp.float32),
            pltpu.VMEM((16, 128), jnp.float32),
            pltpu.SemaphoreType.DMA((NCHUNK,)),
            pltpu.SemaphoreType.DMA((2,)),
            pltpu.SemaphoreType.DMA((NCHUNK,)),
            pltpu.SemaphoreType.DMA((NCHUNK,)),
        ],
        compiler_params=pltpu.CompilerParams(collective_id=0),
    )(x2, g2, b2)


# device time: 7127 ns/iter; 1.0543x vs baseline; 1.0317x over previous
import jax
import jax.numpy as jnp
from jax import lax
from jax.experimental import pallas as pl
from jax.experimental.pallas import tpu as pltpu

N_GLOBAL = 1024
EPS = 1e-5
NCHUNK = 2


def kernel(x, gamma, beta):
    m, n = x.shape
    cm = m // NCHUNK
    r = cm // 128

    def body(
        x_hbm,
        g_hbm,
        b_hbm,
        out_ref,
        xbuf,
        gbuf,
        bbuf,
        stats_ref,
        recv_ref,
        in_sems,
        gb_sems,
        send_sems,
        recv_sems,
    ):
        my_x = lax.axis_index("x")
        my_y = lax.axis_index("y")
        nbr = (my_x, 1 - my_y)

        g_cp = pltpu.make_async_copy(g_hbm, gbuf, gb_sems.at[0])
        g_cp.start()
        b_cp = pltpu.make_async_copy(b_hbm, bbuf, gb_sems.at[1])
        b_cp.start()
        in_copies = []
        for c in range(NCHUNK):
            cp = pltpu.make_async_copy(
                x_hbm.at[c * cm : (c + 1) * cm, :], xbuf.at[c], in_sems.at[c]
            )
            cp.start()
            in_copies.append(cp)

        barrier_sem = pltpu.get_barrier_semaphore()
        pl.semaphore_signal(
            barrier_sem, inc=1, device_id=nbr, device_id_type=pl.DeviceIdType.MESH
        )

        rdmas = []
        for c in range(NCHUNK):
            in_copies[c].wait()
            xa = xbuf[c]
            s = jnp.sum(xa, axis=1)
            q = jnp.sum(xa * xa, axis=1)
            base = c * 2 * r
            stats_ref[base : base + r, :] = s.reshape(r, 128)
            stats_ref[base + r : base + 2 * r, :] = q.reshape(r, 128)
            if c == 0:
                pl.semaphore_wait(barrier_sem, 1)
            rdma = pltpu.make_async_remote_copy(
                src_ref=stats_ref.at[base : base + 2 * r],
                dst_ref=recv_ref.at[base : base + 2 * r],
                send_sem=send_sems.at[c],
                recv_sem=recv_sems.at[c],
                device_id=nbr,
                device_id_type=pl.DeviceIdType.MESH,
            )
            rdma.start()
            rdmas.append(rdma)

        lane = lax.broadcasted_iota(jnp.int32, (128, 128), 1)
        sub = lax.broadcasted_iota(jnp.int32, (128, 128), 0)
        diag = lane == sub
        g_cp.wait()
        b_cp.wait()
        gv = gbuf[0:1, :]
        bv = bbuf[0:1, :]

        for c in range(NCHUNK):
            rdmas[c].wait()
            base = c * 2 * r
            tot = stats_ref[base : base + 2 * r, :] + recv_ref[base : base + 2 * r, :]
            meanl = tot[0:r, :] / N_GLOBAL
            invl = lax.rsqrt(tot[r : 2 * r, :] / N_GLOBAL - meanl * meanl + EPS)
            offl = -meanl * invl
            for a in range(r):
                inv_col = jnp.sum(
                    jnp.where(diag, jnp.broadcast_to(invl[a : a + 1, :], (128, 128)), 0.0),
                    axis=1,
                    keepdims=True,
                )
                off_col = jnp.sum(
                    jnp.where(diag, jnp.broadcast_to(offl[a : a + 1, :], (128, 128)), 0.0),
                    axis=1,
                    keepdims=True,
                )
                lo, hi = a * 128, (a + 1) * 128
                xa = xbuf[c, lo:hi, :]
                out_ref[c * cm + lo : c * cm + hi, :] = (
                    gv * (xa * inv_col + off_col) + bv
                )

    g2 = pltpu.with_memory_space_constraint(
        gamma.reshape(1, n), pltpu.MemorySpace.HBM
    )
    b2 = pltpu.with_memory_space_constraint(
        beta.reshape(1, n), pltpu.MemorySpace.HBM
    )
    x2 = pltpu.with_memory_space_constraint(x, pltpu.MemorySpace.HBM)

    return pl.pallas_call(
        body,
        out_shape=jax.ShapeDtypeStruct((m, n), x.dtype),
        in_specs=[
            pl.BlockSpec(memory_space=pl.ANY),
            pl.BlockSpec(memory_space=pl.ANY),
            pl.BlockSpec(memory_space=pl.ANY),
        ],
        out_specs=pl.BlockSpec(memory_space=pltpu.VMEM),
        scratch_shapes=[
            pltpu.VMEM((NCHUNK, cm, n), jnp.float32),
            pltpu.VMEM((1, n), jnp.float32),
            pltpu.VMEM((1, n), jnp.float32),
            pltpu.VMEM((16, 128), jnp.float32),
            pltpu.VMEM((16, 128), jnp.float32),
            pltpu.SemaphoreType.DMA((NCHUNK,)),
            pltpu.SemaphoreType.DMA((2,)),
            pltpu.SemaphoreType.DMA((NCHUNK,)),
            pltpu.SemaphoreType.DMA((NCHUNK,)),
        ],
        compiler_params=pltpu.CompilerParams(collective_id=0),
    )(x2, g2, b2)
